# Initial kernel scaffold; baseline (speedup 1.0000x reference)
import jax
import jax.numpy as jnp
from jax.experimental import pallas as pl


def kernel(states, actions, agent_qs, max_filter, W1w, W1b, B1w, B1b, Wfw, Wfb, V1w, V1b, V2w, V2b, target):
    raise NotImplementedError("write your pallas kernel here")


# speedup vs baseline: 12.6566x; 12.6566x over previous
import contextlib

import numpy as np
import jax
import jax.numpy as jnp
from jax.experimental import pallas as pl
from jax.experimental.pallas import tpu as pltpu

B, T, N, SD, E, S = 32, 32, 8, 128, 32, 32
BS = B * T
SN = S * N


def _coalition_constants():
    try:
        ctx = jax.default_device(jax.devices("cpu")[0])
    except Exception:
        ctx = contextlib.nullcontext()
    with ctx:
        u = jax.random.uniform(jax.random.key(42), (BS * S, N))
        gc = np.asarray(jnp.argsort(u, axis=1))
    pinv = np.argsort(gc, axis=1)
    den = np.maximum(gc, 1).astype(np.float32)
    mask = (pinv[:, None, :] < gc[:, :, None]).astype(np.float32)
    L = mask / den[:, :, None]
    L = L.reshape(BS, SN, N).transpose(2, 0, 1).copy()
    sel = np.zeros((SN, N), np.float32)
    for i in range(N):
        sel[i::N, i] = 1.0 / S
    return L, sel


_L_CONST, _SEL_CONST = _coalition_constants()


def _mixer_kernel(states_ref, aq_ref, mf_ref, wcat_ref, bcat_ref, v2w_ref,
                  v2b_ref, tgt_ref, L_ref, sel_ref, out_ref, west_ref):
    st = states_ref[:, :]
    n = float(BS)
    ssum = jnp.sum(st, axis=0, keepdims=True)
    ssq = jnp.sum(st * st, axis=0, keepdims=True)
    bm = ssum / n
    bv = (ssq - n * bm * bm) / (n - 1.0)
    c0 = 1e-4
    tot = c0 + n
    new_mean = bm * n / tot
    m2 = 1.0 * c0 + bv * n + bm * bm * c0 * n / tot
    new_var = m2 / tot
    rs = (st - new_mean) * jax.lax.rsqrt(new_var)

    hyper = jnp.dot(rs, wcat_ref[:, :],
                    preferred_element_type=jnp.float32) + bcat_ref[:, :]
    w1a = jnp.abs(hyper[:, 0:E])
    w1b = jnp.abs(hyper[:, E:2 * E])
    b1 = hyper[:, 2 * E:3 * E]
    wf = jnp.abs(hyper[:, 3 * E:4 * E])
    vh = jnp.maximum(hyper[:, 4 * E:5 * E], 0.0)
    v = jnp.dot(vh, v2w_ref[:, :],
                preferred_element_type=jnp.float32) + v2b_ref[:, :]

    aq = aq_ref[:, :]
    nv = L_ref[0] * aq[:, 0:1]
    for a in range(1, N):
        nv = nv + L_ref[a] * aq[:, a:a + 1]
    qb = jnp.concatenate([aq] * S, axis=1)

    acc = jnp.zeros((BS, SN), jnp.float32)
    for e in range(E):
        p = nv * w1a[:, e:e + 1] + qb * w1b[:, e:e + 1] + b1[:, e:e + 1]
        h = jnp.where(p > 0, p, jnp.exp(p) - 1.0)
        acc = acc + h * wf[:, e:e + 1]
    y = jnp.abs(acc + v)

    west = jnp.dot(y, sel_ref[:, :],
                   preferred_element_type=jnp.float32) + 1.0
    west_ref[:, :] = west

    mf = mf_ref[:, :]
    out = jnp.sum((west * (1.0 - mf) + mf) * aq, axis=1, keepdims=True)
    qsum = jnp.sum(aq, axis=1, keepdims=True)
    tgt = tgt_ref[:, :].astype(jnp.float32)
    out_ref[:, :] = jnp.where(tgt != 0.0, qsum, out)


def kernel(states, actions, agent_qs, max_filter, W1w, W1b, B1w, B1b,
           Wfw, Wfb, V1w, V1b, V2w, V2b, target):
    L, sel = _L_CONST, _SEL_CONST
    wcat = jnp.concatenate([W1w, B1w, Wfw, V1w], axis=0).T
    bcat = jnp.concatenate([W1b, B1b, Wfb, V1b]).reshape(1, 5 * E)
    v2w = V2w.T
    v2b = V2b.reshape(1, 1)
    tgt = jnp.asarray(target, jnp.int32).reshape(1, 1)

    out, west = pl.pallas_call(
        _mixer_kernel,
        out_shape=(
            jax.ShapeDtypeStruct((BS, 1), jnp.float32),
            jax.ShapeDtypeStruct((BS, N), jnp.float32),
        ),
    )(states.reshape(BS, SD), agent_qs.reshape(BS, N),
      max_filter.reshape(BS, N), wcat, bcat, v2w, v2b, tgt,
      jnp.asarray(L), jnp.asarray(sel))

    return out.reshape(B, T, 1), west.reshape(B, T, N)


# speedup vs baseline: 12.6567x; 1.0000x over previous
import numpy as np
import jax
import jax.numpy as jnp
from jax.experimental import pallas as pl
from jax.experimental.pallas import tpu as pltpu

B, T, N, SD, E, S = 32, 32, 8, 128, 32, 32
BS = B * T
SN = S * N


def _threefry2x32(k0, k1, x0, x1):
    rot = ((13, 15, 26, 6), (17, 29, 16, 24))
    ks = [np.uint32(k0), np.uint32(k1),
          np.uint32(k0) ^ np.uint32(k1) ^ np.uint32(0x1BD11BDA)]
    x0 = (x0 + ks[0]).astype(np.uint32)
    x1 = (x1 + ks[1]).astype(np.uint32)
    for i in range(5):
        for r in rot[i % 2]:
            x0 = (x0 + x1).astype(np.uint32)
            x1 = ((x1 << np.uint32(r)) | (x1 >> np.uint32(32 - r))).astype(np.uint32)
            x1 = x1 ^ x0
        x0 = (x0 + ks[(i + 1) % 3]).astype(np.uint32)
        x1 = (x1 + ks[(i + 2) % 3] + np.uint32(i + 1)).astype(np.uint32)
    return x0, x1


def _uniform_key42(shape):
    size = int(np.prod(shape))
    counts = np.arange(size, dtype=np.uint32)
    b0, b1 = _threefry2x32(0, 42, np.zeros(size, np.uint32), counts)
    bits = (b0 ^ b1).reshape(shape)
    f = ((bits >> np.uint32(9)) | np.uint32(0x3F800000)).view(np.float32)
    return np.maximum(0.0, f - 1.0).astype(np.float32)


def _coalition_constants():
    u = _uniform_key42((BS * S, N))
    gc = np.argsort(u, axis=1, kind="stable")
    pinv = np.argsort(gc, axis=1)
    den = np.maximum(gc, 1).astype(np.float32)
    mask = (pinv[:, None, :] < gc[:, :, None]).astype(np.float32)
    L = mask / den[:, :, None]
    L = L.reshape(BS, SN, N).transpose(2, 0, 1).copy()
    sel = np.zeros((SN, N), np.float32)
    for i in range(N):
        sel[i::N, i] = 1.0 / S
    return L, sel


_L_CONST, _SEL_CONST = _coalition_constants()


def _mixer_kernel(states_ref, aq_ref, mf_ref, wcat_ref, bcat_ref, v2w_ref,
                  v2b_ref, tgt_ref, L_ref, sel_ref, out_ref, west_ref):
    st = states_ref[:, :]
    n = float(BS)
    ssum = jnp.sum(st, axis=0, keepdims=True)
    ssq = jnp.sum(st * st, axis=0, keepdims=True)
    bm = ssum / n
    bv = (ssq - n * bm * bm) / (n - 1.0)
    c0 = 1e-4
    tot = c0 + n
    new_mean = bm * n / tot
    m2 = 1.0 * c0 + bv * n + bm * bm * c0 * n / tot
    new_var = m2 / tot
    rs = (st - new_mean) * jax.lax.rsqrt(new_var)

    hyper = jnp.dot(rs, wcat_ref[:, :],
                    preferred_element_type=jnp.float32) + bcat_ref[:, :]
    w1a = jnp.abs(hyper[:, 0:E])
    w1b = jnp.abs(hyper[:, E:2 * E])
    b1 = hyper[:, 2 * E:3 * E]
    wf = jnp.abs(hyper[:, 3 * E:4 * E])
    vh = jnp.maximum(hyper[:, 4 * E:5 * E], 0.0)
    v = jnp.dot(vh, v2w_ref[:, :],
                preferred_element_type=jnp.float32) + v2b_ref[:, :]

    aq = aq_ref[:, :]
    nv = L_ref[0] * aq[:, 0:1]
    for a in range(1, N):
        nv = nv + L_ref[a] * aq[:, a:a + 1]
    qb = jnp.concatenate([aq] * S, axis=1)

    acc = jnp.zeros((BS, SN), jnp.float32)
    for e in range(E):
        p = nv * w1a[:, e:e + 1] + qb * w1b[:, e:e + 1] + b1[:, e:e + 1]
        h = jnp.where(p > 0, p, jnp.exp(p) - 1.0)
        acc = acc + h * wf[:, e:e + 1]
    y = jnp.abs(acc + v)

    west = jnp.dot(y, sel_ref[:, :],
                   preferred_element_type=jnp.float32) + 1.0
    west_ref[:, :] = west

    mf = mf_ref[:, :]
    out = jnp.sum((west * (1.0 - mf) + mf) * aq, axis=1, keepdims=True)
    qsum = jnp.sum(aq, axis=1, keepdims=True)
    tgt = tgt_ref[:, :].astype(jnp.float32)
    out_ref[:, :] = jnp.where(tgt != 0.0, qsum, out)


def kernel(states, actions, agent_qs, max_filter, W1w, W1b, B1w, B1b,
           Wfw, Wfb, V1w, V1b, V2w, V2b, target):
    L, sel = _L_CONST, _SEL_CONST
    wcat = jnp.concatenate([W1w, B1w, Wfw, V1w], axis=0).T
    bcat = jnp.concatenate([W1b, B1b, Wfb, V1b]).reshape(1, 5 * E)
    v2w = V2w.T
    v2b = V2b.reshape(1, 1)
    tgt = jnp.asarray(target, jnp.int32).reshape(1, 1)

    out, west = pl.pallas_call(
        _mixer_kernel,
        out_shape=(
            jax.ShapeDtypeStruct((BS, 1), jnp.float32),
            jax.ShapeDtypeStruct((BS, N), jnp.float32),
        ),
    )(states.reshape(BS, SD), agent_qs.reshape(BS, N),
      max_filter.reshape(BS, N), wcat, bcat, v2w, v2b, tgt,
      jnp.asarray(L), jnp.asarray(sel))

    return out.reshape(B, T, 1), west.reshape(B, T, N)
